# baseline (speedup 1.0000x reference)
import functools
import math

import jax
import jax.numpy as jnp
from jax import lax
from jax.experimental import pallas as pl
from jax.experimental.pallas import tpu as pltpu
from jax.experimental.pallas import tpu_sc as plsc

D_MODEL_ = 256
POS_DIM_ = 32
_CH = 256
_PADROWS = 8
_CHUNK = 64


def _fourier_table(M, dtype):
    half = POS_DIM_ // 2
    freqs = jnp.exp(jnp.linspace(math.log(1.0), math.log(1000.0), half))
    idx = jnp.arange(M, dtype=dtype)
    pos = jnp.clip(idx / max(1, M - 1), 0.0, 1.0)
    ang = pos[:, None] * freqs
    out = jnp.concatenate([jnp.sin(ang), jnp.cos(ang)], axis=-1)
    if out.shape[-1] < POS_DIM_:
        out = jnp.concatenate(
            [out, jnp.zeros(out.shape[:-1] + (POS_DIM_ - out.shape[-1],), out.dtype)],
            axis=-1)
    return out.astype(dtype)


def _tc_body(x_ref, w_ref, b_ref, ff_ref, q_ref, c_ref):
    T = x_ref.shape[1]
    D = D_MODEL_
    i = pl.program_id(0)

    xb = x_ref[0].astype(jnp.bfloat16)
    w_top = w_ref[:D, :].astype(jnp.bfloat16)
    y = jnp.dot(xb, w_top, preferred_element_type=jnp.float32)

    r16 = jax.lax.broadcasted_iota(jnp.int16, (_CH, _CH), 0)
    c16 = jax.lax.broadcasted_iota(jnp.int16, (_CH, _CH), 1)
    tri = jnp.where(r16 >= c16, jnp.bfloat16(1.0), jnp.bfloat16(0.0))

    carry = jnp.zeros((1, D), jnp.float32)
    for ch in range(T // _CH):
        blk = y[ch * _CH:(ch + 1) * _CH, :].astype(jnp.bfloat16)
        pc = jnp.dot(tri, blk, preferred_element_type=jnp.float32)
        q_ref[0, ch * _CH:(ch + 1) * _CH, :] = pc + carry
        carry = carry + pc[_CH - 1:_CH, :]
    q_ref[0, T:T + _PADROWS, :] = jnp.zeros((_PADROWS, D), jnp.float32)

    @pl.when(i == 0)
    def _const():
        w_bot = w_ref[D:, :]
        c_ref[...] = (jnp.dot(ff_ref[...], w_bot,
                              preferred_element_type=jnp.float32)
                      + b_ref[...][None, :])


def _make_sc_gather(n_items, D, n_workers):
    per_w = n_items // n_workers
    n_chunks = per_w // _CHUNK
    mesh = plsc.VectorSubcoreMesh(core_axis_name="c", subcore_axis_name="s")

    @functools.partial(
        pl.kernel, mesh=mesh,
        out_type=jax.ShapeDtypeStruct((n_items, D), jnp.float32),
        scratch_types=[
            pltpu.VMEM((_CHUNK,), jnp.int32),
            pltpu.VMEM((_CHUNK,), jnp.int32),
            pltpu.VMEM((_CHUNK, D), jnp.float32),
            pltpu.VMEM((_CHUNK, D), jnp.float32),
            pltpu.VMEM((_CHUNK, 16), jnp.float32),
            pltpu.VMEM((_CHUNK, D), jnp.float32),
            pltpu.SemaphoreType.DMA,
            pltpu.SemaphoreType.DMA,
        ],
    )
    def sc_gather(q_hbm, idxe_hbm, idxs_hbm, inv_hbm, c_hbm, out_hbm,
                  idxe_v, idxs_v, re_v, rs_v, inv_v, c_v, sem_e, sem_s):
        nc = jax.lax.axis_size("c")
        wid = lax.axis_index("s") * nc + lax.axis_index("c")
        base = wid * per_w
        for k in range(n_chunks):
            off = base + k * _CHUNK
            pltpu.sync_copy(idxe_hbm.at[pl.ds(off, _CHUNK)], idxe_v)
            pltpu.sync_copy(idxs_hbm.at[pl.ds(off, _CHUNK)], idxs_v)
            pltpu.sync_copy(inv_hbm.at[pl.ds(off, _CHUNK), :], inv_v)
            pltpu.sync_copy(c_hbm.at[pl.ds(off, _CHUNK), :], c_v)
            pltpu.async_copy(q_hbm.at[idxe_v], re_v, sem_e).wait()
            pltpu.async_copy(q_hbm.at[idxs_v], rs_v, sem_s).wait()

            def body(r, _):
                invv = inv_v[r, :]
                for j in range(D // 16):
                    sl = pl.ds(j * 16, 16)
                    re_v[r, sl] = ((re_v[r, sl] - rs_v[r, sl]) * invv
                                   + c_v[r, sl])
                return 0

            lax.fori_loop(0, _CHUNK, body, 0)
            pltpu.sync_copy(re_v, out_hbm.at[pl.ds(off, _CHUNK), :])

    return sc_gather


def kernel(frame_emb, beat_bounds, W, b):
    B, T, D = frame_emb.shape
    M = beat_bounds.shape[1]
    Tp = T + _PADROWS
    bounds = beat_bounds.astype(jnp.int32)
    ff = _fourier_table(M, frame_emb.dtype)

    q, c_tab = pl.pallas_call(
        _tc_body,
        grid=(B,),
        in_specs=[
            pl.BlockSpec((1, T, D), lambda i: (i, 0, 0)),
            pl.BlockSpec((D + POS_DIM_, D), lambda i: (0, 0)),
            pl.BlockSpec((D,), lambda i: (0,)),
            pl.BlockSpec((M, POS_DIM_), lambda i: (0, 0)),
        ],
        out_specs=[
            pl.BlockSpec((1, Tp, D), lambda i: (i, 0, 0)),
            pl.BlockSpec((M, D), lambda i: (0, 0)),
        ],
        out_shape=[
            jax.ShapeDtypeStruct((B, Tp, D), jnp.float32),
            jax.ShapeDtypeStruct((M, D), jnp.float32),
        ],
        compiler_params=pltpu.CompilerParams(
            dimension_semantics=("arbitrary",)),
    )(frame_emb, W, b, ff)

    s = jnp.clip(bounds[..., 0], 0, T - 1)
    e = jnp.minimum(bounds[..., 1], T)
    e = jnp.maximum(s + 1, e)
    cnt = (e - s).astype(jnp.float32)
    brow = jnp.arange(B, dtype=jnp.int32)[:, None] * Tp
    idx_e = (brow + e - 1).reshape(-1)
    idx_s = jnp.where(s > 0, brow + s - 1, brow + T).reshape(-1)
    inv_tile = jnp.broadcast_to((1.0 / cnt).reshape(-1)[:, None],
                                (B * M, 16)).astype(jnp.float32)

    info = plsc.get_sparse_core_info()
    n_workers = info.num_cores * info.num_subcores
    sc_gather = _make_sc_gather(B * M, D, n_workers)
    c_full = jnp.broadcast_to(c_tab[None, :, :], (B, M, D)).reshape(B * M, D)
    out = sc_gather(q.reshape(B * Tp, D), idx_e, idx_s, inv_tile, c_full)
    return out.reshape(B, M, D)


# speedup vs baseline: 2.5879x; 2.5879x over previous
import math

import jax
import jax.numpy as jnp
from jax.experimental import pallas as pl

D_MODEL_ = 256
POS_DIM_ = 32


def _fourier_table(M, dtype):
    half = POS_DIM_ // 2
    freqs = jnp.exp(jnp.linspace(math.log(1.0), math.log(1000.0), half))
    idx = jnp.arange(M, dtype=dtype)
    pos = jnp.clip(idx / max(1, M - 1), 0.0, 1.0)
    ang = pos[:, None] * freqs
    out = jnp.concatenate([jnp.sin(ang), jnp.cos(ang)], axis=-1)
    if out.shape[-1] < POS_DIM_:
        out = jnp.concatenate(
            [out, jnp.zeros(out.shape[:-1] + (POS_DIM_ - out.shape[-1],), out.dtype)],
            axis=-1)
    return out.astype(dtype)


def _pool_kernel(bounds_ref, x_ref, w_ref, bias_ref, ff_ref, o_ref):
    T = x_ref.shape[1]
    M = bounds_ref.shape[1]
    s = bounds_ref[0, :, 0]
    e = bounds_ref[0, :, 1]
    s = jnp.clip(s, 0, T - 1)
    e = jnp.minimum(e, T)
    e = jnp.maximum(s + 1, e)

    t = jax.lax.broadcasted_iota(jnp.int32, (M, T), 1)
    mask = (t >= s[:, None]) & (t < e[:, None])
    maskf = mask.astype(jnp.float32)

    sums = jnp.dot(maskf, x_ref[0], preferred_element_type=jnp.float32)
    inv = 1.0 / (e - s).astype(jnp.float32)
    mean = sums * inv[:, None]

    w_top = w_ref[:D_MODEL_, :]
    w_bot = w_ref[D_MODEL_:, :]
    out = jnp.dot(mean, w_top, preferred_element_type=jnp.float32)
    out += jnp.dot(ff_ref[...], w_bot, preferred_element_type=jnp.float32)
    out += bias_ref[...][None, :]
    o_ref[0] = out


def kernel(frame_emb, beat_bounds, W, b):
    B, T, D = frame_emb.shape
    M = beat_bounds.shape[1]
    bounds = beat_bounds.astype(jnp.int32)
    ff = _fourier_table(M, frame_emb.dtype)

    return pl.pallas_call(
        _pool_kernel,
        grid=(B,),
        in_specs=[
            pl.BlockSpec((1, M, 2), lambda i: (i, 0, 0)),
            pl.BlockSpec((1, T, D), lambda i: (i, 0, 0)),
            pl.BlockSpec((D + POS_DIM_, D), lambda i: (0, 0)),
            pl.BlockSpec((D,), lambda i: (0,)),
            pl.BlockSpec((M, POS_DIM_), lambda i: (0, 0)),
        ],
        out_specs=pl.BlockSpec((1, M, D), lambda i: (i, 0, 0)),
        out_shape=jax.ShapeDtypeStruct((B, M, D), frame_emb.dtype),
    )(bounds, frame_emb, W, b, ff)
